# Initial kernel scaffold; baseline (speedup 1.0000x reference)
import jax
import jax.numpy as jnp
from jax.experimental import pallas as pl


def kernel(x, edge_index, edge_attr, batch, W1_rel, b1, W1_root, W2_rel, b2, W2_root, Wl1, bl1, Wl2, bl2, Wl3, bl3):
    raise NotImplementedError("write your pallas kernel here")


# speedup vs baseline: 2.3011x; 2.3011x over previous
import functools

import jax
import jax.numpy as jnp
from jax import lax
from jax.experimental import pallas as pl
from jax.experimental.pallas import tpu as pltpu
from jax.experimental.pallas import tpu_sc as plsc

_N = 10000
_E = 320000
_F = 128
_H = 512
_G = 64

_NC = 2
_NS = 16
_NW = _NC * _NS
_K = 128
_CHUNKS = -(-_E // (_NW * _K))
_EPT = _K * _CHUNKS
_EPAD = _NW * _EPT
_NPAD = 10240
_RPT = _NPAD // _NS

_BM = 1000


@functools.cache
def _get_mp_pass():
    return pl.kernel(
        _mp_body,
        out_type=jax.ShapeDtypeStruct((_NC, _NPAD, _F), jnp.float32),
        mesh=plsc.VectorSubcoreMesh(core_axis_name="c", subcore_axis_name="s",
                                    num_cores=_NC, num_subcores=_NS),
        scratch_types=[
            pltpu.VMEM_SHARED((_NPAD, _F), jnp.float32),
            pltpu.VMEM((_K,), jnp.int32),
            pltpu.VMEM((_K,), jnp.int32),
            pltpu.VMEM((_K,), jnp.float32),
            pltpu.VMEM((_K, _F), jnp.float32),
            pltpu.SemaphoreType.DMA,
        ],
    )


def _mp_body(table, src, dst, w, zeros, out, accum, src_v, dst_v, w_v,
             rows_v, sem):
    cid = lax.axis_index("c")
    sid = lax.axis_index("s")

    pltpu.sync_copy(zeros, accum.at[pl.ds(sid * _RPT, _RPT)])
    plsc.subcore_barrier()

    ebase = cid * (_NS * _EPT) + sid * _EPT

    @pl.loop(0, _CHUNKS)
    def _chunk(i):
        base = ebase + i * _K
        pltpu.sync_copy(src.at[pl.ds(base, _K)], src_v)
        pltpu.sync_copy(dst.at[pl.ds(base, _K)], dst_v)
        pltpu.sync_copy(w.at[pl.ds(base, _K)], w_v)
        pltpu.async_copy(table.at[src_v], rows_v, sem).wait()

        @plsc.parallel_loop(0, _K // 16)
        def _grp(g):
            wvec = w_v[pl.ds(g * 16, 16)]
            for j in range(16):
                wk = wvec[j]
                k = g * 16 + j
                for v in range(_F // 16):
                    sl = pl.ds(v * 16, 16)
                    rows_v[k, sl] = rows_v[k, sl] * wk

        pltpu.sync_copy(rows_v, accum.at[dst_v], add=True)

    plsc.subcore_barrier()
    pltpu.sync_copy(accum.at[pl.ds(sid * _RPT, _RPT)],
                    out.at[cid, pl.ds(sid * _RPT, _RPT)])


def _dense1_body(p_ref, x_ref, wrel_ref, b_ref, wroot_ref, o0, o1, o2, o3):
    s = p_ref[0] + p_ref[1]
    xb = x_ref[...]
    outs = (o0, o1, o2, o3)
    for c in range(4):
        wslice = slice(c * _F, (c + 1) * _F)
        acc = jnp.dot(s, wrel_ref[:, wslice], preferred_element_type=jnp.float32)
        acc = acc + jnp.dot(xb, wroot_ref[:, wslice],
                            preferred_element_type=jnp.float32)
        acc = acc + b_ref[0, wslice][None, :]
        outs[c][...] = jnp.maximum(acc, 0.0)


def _dense1(p, x, w_rel, b, w_root):
    return pl.pallas_call(
        _dense1_body,
        grid=(_N // _BM,),
        in_specs=[
            pl.BlockSpec((_NC, _BM, _F), lambda m: (0, m, 0)),
            pl.BlockSpec((_BM, _F), lambda m: (m, 0)),
            pl.BlockSpec((_F, _H), lambda m: (0, 0)),
            pl.BlockSpec((1, _H), lambda m: (0, 0)),
            pl.BlockSpec((_F, _H), lambda m: (0, 0)),
        ],
        out_specs=[pl.BlockSpec((_BM, _F), lambda m: (m, 0))] * 4,
        out_shape=[jax.ShapeDtypeStruct((_N, _F), jnp.float32)] * 4,
    )(p, x, w_rel, b.reshape(1, _H), w_root)


def _dense2_body(q0, q1, q2, q3, h0, h1, h2r, h3, wrel_ref, b_ref, wroot_ref,
                 bt_ref, wl1, bl1, wl2, bl2, wl3, bl3, out_ref,
                 pooled, counts):
    m = pl.program_id(0)
    nblocks = pl.num_programs(0)
    qs = (q0, q1, q2, q3)
    hs = (h0, h1, h2r, h3)

    acc = jnp.broadcast_to(b_ref[0][None, :], (_BM, _H))
    for c in range(4):
        ksl = slice(c * _F, (c + 1) * _F)
        aggc = qs[c][0] + qs[c][1]
        acc = acc + jnp.dot(aggc, wrel_ref[ksl, :],
                            preferred_element_type=jnp.float32)
        acc = acc + jnp.dot(hs[c][...], wroot_ref[ksl, :],
                            preferred_element_type=jnp.float32)
    hout = jnp.maximum(acc, 0.0)

    bt = bt_ref[0, 0, :]
    onehot_t = (lax.broadcasted_iota(jnp.int32, (128, _BM), 0)
                == bt[None, :]).astype(jnp.float32)

    @pl.when(m == 0)
    def _init():
        pooled[...] = jnp.zeros_like(pooled)
        counts[...] = jnp.zeros_like(counts)

    pooled[...] += jnp.dot(onehot_t, hout, preferred_element_type=jnp.float32)
    counts[...] += jnp.broadcast_to(
        jnp.sum(onehot_t, axis=1, keepdims=True), (128, 128))

    @pl.when(m == nblocks - 1)
    def _final():
        cnt = counts[:, 0:1]
        mean = pooled[...] / jnp.maximum(cnt, 1.0)
        r = jnp.maximum(jnp.dot(mean, wl1[...],
                                preferred_element_type=jnp.float32)
                        + bl1[0][None, :], 0.0)
        r = jnp.maximum(jnp.dot(r, wl2[...],
                                preferred_element_type=jnp.float32)
                        + bl2[0][None, :], 0.0)
        o = jnp.dot(r, wl3[...], preferred_element_type=jnp.float32) \
            + bl3[0][None, :]
        out_ref[...] = jnp.broadcast_to(o[:_G, :], (_G, 128))


def _dense2(q, h1s, w_rel, b, w_root, bt3, wl1, bl1, wl2, bl2, wl3, bl3):
    full2 = lambda a, b_: pl.BlockSpec((a, b_), lambda m: (0, 0))
    return pl.pallas_call(
        _dense2_body,
        grid=(_N // _BM,),
        in_specs=[
            *[pl.BlockSpec((_NC, _BM, _F), lambda m: (0, m, 0))] * 4,
            *[pl.BlockSpec((_BM, _F), lambda m: (m, 0))] * 4,
            full2(_H, _H),
            full2(1, _H),
            full2(_H, _H),
            pl.BlockSpec((1, 1, _BM), lambda m: (m, 0, 0)),
            full2(_H, 64),
            full2(1, 64),
            full2(64, 16),
            full2(1, 16),
            full2(16, 1),
            full2(1, 1),
        ],
        out_specs=pl.BlockSpec((_G, 128), lambda m: (0, 0)),
        out_shape=jax.ShapeDtypeStruct((_G, 128), jnp.float32),
        scratch_shapes=[
            pltpu.VMEM((128, _H), jnp.float32),
            pltpu.VMEM((128, 128), jnp.float32),
        ],
    )(*q, *h1s, w_rel, b.reshape(1, _H), w_root, bt3,
      wl1, bl1.reshape(1, 64), wl2, bl2.reshape(1, 16),
      wl3, bl3.reshape(1, 1))


def kernel(x, edge_index, edge_attr, batch, W1_rel, b1, W1_root, W2_rel, b2,
           W2_root, Wl1, bl1, Wl2, bl2, Wl3, bl3):
    pad = _EPAD - _E
    src = jnp.pad(edge_index[0], (0, pad))
    dst = jnp.pad(edge_index[1], (0, pad))
    w = jnp.pad(edge_attr, (0, pad))
    zeros = jnp.zeros((_RPT, _F), jnp.float32)

    mp = _get_mp_pass()
    p = mp(x, src, dst, w, zeros)
    h1s = _dense1(p, x, W1_rel, b1, W1_root)
    q = [mp(h1s[c], src, dst, w, zeros) for c in range(4)]
    bt3 = batch.reshape(_N // _BM, 1, _BM)
    out = _dense2(q, h1s, W2_rel, b2, W2_root, bt3,
                  Wl1, bl1, Wl2, bl2, Wl3, bl3)
    return out[:, :1]
